# baseline (speedup 1.0000x reference)
import functools

import jax
import jax.numpy as jnp
from jax.experimental import pallas as pl
from jax.experimental.pallas import tpu as pltpu

NB = 4
BLOCK_K = 2048

_DN = (((1,), (1,)), ((), ()))


def _bf16_dot(a_ref, w_ref):
    a = a_ref[...].astype(jnp.bfloat16)
    w = w_ref[...].astype(jnp.bfloat16)
    return jax.lax.dot_general(a, w, _DN, preferred_element_type=jnp.float32)


def _nnue_body(*refs, block_b, has_tail):
    (pov_ref, w_ref, bk_ref, waW_ref, wab_ref, baW_ref, bab_ref,
     f0W_ref, f0b_ref, f1W_ref, f1b_ref, f2W_ref, f2b_ref) = refs[:13]
    if has_tail:
        wt_ref, bt_ref, waWt_ref, baWt_ref = refs[13:17]
        out_ref, accw_ref, accb_ref = refs[17:]
    else:
        out_ref, accw_ref, accb_ref = refs[13:]

    k = pl.program_id(0)
    b = pl.program_id(1)
    nk = pl.num_programs(0)

    pw = _bf16_dot(w_ref, waW_ref)
    pb = _bf16_dot(bk_ref, baW_ref)

    rows = pl.ds(b * block_b, block_b)

    @pl.when(k == 0)
    def _init():
        if has_tail:
            accw_ref[rows, :] = pw + _bf16_dot(wt_ref, waWt_ref)
            accb_ref[rows, :] = pb + _bf16_dot(bt_ref, baWt_ref)
        else:
            accw_ref[rows, :] = pw
            accb_ref[rows, :] = pb

    @pl.when(k > 0)
    def _accum():
        accw_ref[rows, :] += pw
        accb_ref[rows, :] += pb

    @pl.when(k == nk - 1)
    def _epilogue():
        w256 = accw_ref[rows, :] + wab_ref[...][None, :]
        b256 = accb_ref[rows, :] + bab_ref[...][None, :]
        p = pov_ref[...]
        x0 = jnp.maximum(p * w256 + (1.0 - p) * b256, 0.0)
        x1 = jnp.maximum(p * b256 + (1.0 - p) * w256, 0.0)
        f0 = f0W_ref[...]
        h = f0.shape[1] // 2
        y = (jax.lax.dot_general(x0, f0[:, :h], _DN,
                                 preferred_element_type=jnp.float32)
             + jax.lax.dot_general(x1, f0[:, h:], _DN,
                                   preferred_element_type=jnp.float32)
             + f0b_ref[...][None, :])
        y = jnp.maximum(y, 0.0)
        z = jax.lax.dot_general(y, f1W_ref[...], _DN,
                                preferred_element_type=jnp.float32)
        z = jnp.maximum(z + f1b_ref[...][None, :], 0.0)
        o = jnp.sum(z * f2W_ref[...], axis=1, keepdims=True)
        out_ref[rows, :] = o + f2b_ref[0]


def kernel(pov, white, black, wa_W, wa_b, ba_W, ba_b,
           fc0_W, fc0_b, fc1_W, fc1_b, fc2_W, fc2_b):
    B, K = white.shape
    H = wa_W.shape[0]
    block_b = B // NB
    block_k = min(BLOCK_K, K)
    nk = K // block_k
    k_main = nk * block_k
    tail = K - k_main

    grid = (nk, NB)
    full = lambda arr: pl.BlockSpec(arr.shape, lambda k, b: (0,) * arr.ndim)

    in_specs = [
        pl.BlockSpec((block_b, 1), lambda k, b: (b, 0)),
        pl.BlockSpec((block_b, block_k), lambda k, b: (b, k)),
        pl.BlockSpec((block_b, block_k), lambda k, b: (b, k)),
        pl.BlockSpec((H, block_k), lambda k, b: (0, k)),
        full(wa_b),
        pl.BlockSpec((H, block_k), lambda k, b: (0, k)),
        full(ba_b),
        full(fc0_W), full(fc0_b),
        full(fc1_W), full(fc1_b),
        full(fc2_W),
        pl.BlockSpec(memory_space=pltpu.SMEM),
    ]
    args = [pov, white, black, wa_W, wa_b, ba_W, ba_b,
            fc0_W, fc0_b, fc1_W, fc1_b, fc2_W, fc2_b]
    if tail:
        args += [white[:, k_main:], black[:, k_main:],
                 wa_W[:, k_main:], ba_W[:, k_main:]]
        in_specs += [
            pl.BlockSpec((block_b, tail), lambda k, b: (b, 0)),
            pl.BlockSpec((block_b, tail), lambda k, b: (b, 0)),
            pl.BlockSpec((H, tail), lambda k, b: (0, 0)),
            pl.BlockSpec((H, tail), lambda k, b: (0, 0)),
        ]

    out = pl.pallas_call(
        functools.partial(_nnue_body, block_b=block_b, has_tail=bool(tail)),
        grid=grid,
        in_specs=in_specs,
        out_specs=pl.BlockSpec((B, 1), lambda k, b: (0, 0)),
        out_shape=jax.ShapeDtypeStruct((B, 1), jnp.float32),
        scratch_shapes=[
            pltpu.VMEM((B, H), jnp.float32),
            pltpu.VMEM((B, H), jnp.float32),
        ],
        compiler_params=pltpu.CompilerParams(
            dimension_semantics=("arbitrary", "arbitrary"),
        ),
    )(*args)
    return out


# speedup vs baseline: 1.0720x; 1.0720x over previous
import functools

import jax
import jax.numpy as jnp
from jax.experimental import pallas as pl
from jax.experimental.pallas import tpu as pltpu

NB = 4
BLOCK_K = 4096

_DN = (((1,), (1,)), ((), ()))


def _bf16_dot(a_ref, w_ref):
    a = a_ref[...].astype(jnp.bfloat16)
    w = w_ref[...].astype(jnp.bfloat16)
    return jax.lax.dot_general(a, w, _DN, preferred_element_type=jnp.float32)


def _nnue_body(*refs, block_b, has_tail):
    (pov_ref, w_ref, bk_ref, waW_ref, wab_ref, baW_ref, bab_ref,
     f0W_ref, f0b_ref, f1W_ref, f1b_ref, f2W_ref, f2b_ref) = refs[:13]
    if has_tail:
        wt_ref, bt_ref, waWt_ref, baWt_ref = refs[13:17]
        out_ref, accw_ref, accb_ref = refs[17:]
    else:
        out_ref, accw_ref, accb_ref = refs[13:]

    k = pl.program_id(0)
    b = pl.program_id(1)
    nk = pl.num_programs(0)

    pw = _bf16_dot(w_ref, waW_ref)
    pb = _bf16_dot(bk_ref, baW_ref)

    rows = pl.ds(b * block_b, block_b)

    @pl.when(k == 0)
    def _init():
        if has_tail:
            accw_ref[rows, :] = pw + _bf16_dot(wt_ref, waWt_ref)
            accb_ref[rows, :] = pb + _bf16_dot(bt_ref, baWt_ref)
        else:
            accw_ref[rows, :] = pw
            accb_ref[rows, :] = pb

    @pl.when(k > 0)
    def _accum():
        accw_ref[rows, :] += pw
        accb_ref[rows, :] += pb

    @pl.when(k == nk - 1)
    def _epilogue():
        w256 = accw_ref[rows, :] + wab_ref[...][None, :]
        b256 = accb_ref[rows, :] + bab_ref[...][None, :]
        p = pov_ref[...]
        x0 = jnp.maximum(p * w256 + (1.0 - p) * b256, 0.0)
        x1 = jnp.maximum(p * b256 + (1.0 - p) * w256, 0.0)
        f0 = f0W_ref[...]
        h = f0.shape[1] // 2
        y = (jax.lax.dot_general(x0, f0[:, :h], _DN,
                                 preferred_element_type=jnp.float32)
             + jax.lax.dot_general(x1, f0[:, h:], _DN,
                                   preferred_element_type=jnp.float32)
             + f0b_ref[...][None, :])
        y = jnp.maximum(y, 0.0)
        z = jax.lax.dot_general(y, f1W_ref[...], _DN,
                                preferred_element_type=jnp.float32)
        z = jnp.maximum(z + f1b_ref[...][None, :], 0.0)
        o = jnp.sum(z * f2W_ref[...], axis=1, keepdims=True)
        out_ref[rows, :] = o + f2b_ref[0]


def kernel(pov, white, black, wa_W, wa_b, ba_W, ba_b,
           fc0_W, fc0_b, fc1_W, fc1_b, fc2_W, fc2_b):
    B, K = white.shape
    H = wa_W.shape[0]
    block_b = B // NB
    block_k = min(BLOCK_K, K)
    nk = K // block_k
    k_main = nk * block_k
    tail = K - k_main

    grid = (nk, NB)
    full = lambda arr: pl.BlockSpec(arr.shape, lambda k, b: (0,) * arr.ndim)

    in_specs = [
        pl.BlockSpec((block_b, 1), lambda k, b: (b, 0)),
        pl.BlockSpec((block_b, block_k), lambda k, b: (b, k)),
        pl.BlockSpec((block_b, block_k), lambda k, b: (b, k)),
        pl.BlockSpec((H, block_k), lambda k, b: (0, k)),
        full(wa_b),
        pl.BlockSpec((H, block_k), lambda k, b: (0, k)),
        full(ba_b),
        full(fc0_W), full(fc0_b),
        full(fc1_W), full(fc1_b),
        full(fc2_W),
        pl.BlockSpec(memory_space=pltpu.SMEM),
    ]
    args = [pov, white, black, wa_W, wa_b, ba_W, ba_b,
            fc0_W, fc0_b, fc1_W, fc1_b, fc2_W, fc2_b]
    if tail:
        args += [white[:, k_main:], black[:, k_main:],
                 wa_W[:, k_main:], ba_W[:, k_main:]]
        in_specs += [
            pl.BlockSpec((block_b, tail), lambda k, b: (b, 0)),
            pl.BlockSpec((block_b, tail), lambda k, b: (b, 0)),
            pl.BlockSpec((H, tail), lambda k, b: (0, 0)),
            pl.BlockSpec((H, tail), lambda k, b: (0, 0)),
        ]

    out = pl.pallas_call(
        functools.partial(_nnue_body, block_b=block_b, has_tail=bool(tail)),
        grid=grid,
        in_specs=in_specs,
        out_specs=pl.BlockSpec((B, 1), lambda k, b: (0, 0)),
        out_shape=jax.ShapeDtypeStruct((B, 1), jnp.float32),
        scratch_shapes=[
            pltpu.VMEM((B, H), jnp.float32),
            pltpu.VMEM((B, H), jnp.float32),
        ],
        compiler_params=pltpu.CompilerParams(
            dimension_semantics=("arbitrary", "arbitrary"),
        ),
    )(*args)
    return out
